# baseline (speedup 1.0000x reference)
import functools
import math

import jax
import jax.numpy as jnp
from jax import lax
from jax.experimental import pallas as pl
from jax.experimental.pallas import tpu as pltpu
from jax.experimental.pallas import tpu_sc as plsc

D = 128
NSEG = 200
RATIO = 0.8
EPAD = 327680
CHUNK = 128


def _sc_gather_body(h_hbm, src_hbm, out_hbm,
                    idx0, idx1, rows0, rows1, sem0, sem1):
    nc = 2
    wid = lax.axis_index("s") * nc + lax.axis_index("c")
    rows_per_w = EPAD // 32
    nchunks = rows_per_w // CHUNK
    base = wid * rows_per_w

    pltpu.sync_copy(src_hbm.at[pl.ds(base, CHUNK)], idx0)
    pltpu.async_copy(h_hbm.at[idx0], rows0, sem0)

    def step(it, _):
        g = 2 * it
        pltpu.sync_copy(src_hbm.at[pl.ds(base + (g + 1) * CHUNK, CHUNK)], idx1)
        pltpu.async_copy(h_hbm.at[idx1], rows1, sem1)
        pltpu.make_async_copy(h_hbm.at[idx0], rows0, sem0).wait()
        pltpu.sync_copy(rows0, out_hbm.at[pl.ds(base + g * CHUNK, CHUNK)])
        @pl.when(g + 2 < nchunks)
        def _():
            pltpu.sync_copy(src_hbm.at[pl.ds(base + (g + 2) * CHUNK, CHUNK)], idx0)
            pltpu.async_copy(h_hbm.at[idx0], rows0, sem0)
        pltpu.make_async_copy(h_hbm.at[idx1], rows1, sem1).wait()
        pltpu.sync_copy(rows1, out_hbm.at[pl.ds(base + (g + 1) * CHUNK, CHUNK)])
        return 0

    lax.fori_loop(0, nchunks // 2, step, 0, unroll=False)


def _sc_gather(h, src):
    mesh = plsc.VectorSubcoreMesh(core_axis_name="c", subcore_axis_name="s")
    kern = pl.kernel(
        _sc_gather_body,
        mesh=mesh,
        out_type=jax.ShapeDtypeStruct((EPAD, D), jnp.float32),
        scratch_types=[
            pltpu.VMEM((CHUNK,), jnp.int32),
            pltpu.VMEM((CHUNK,), jnp.int32),
            pltpu.VMEM((CHUNK, D), jnp.float32),
            pltpu.VMEM((CHUNK, D), jnp.float32),
            pltpu.SemaphoreType.DMA,
            pltpu.SemaphoreType.DMA,
        ],
    )
    return kern(h, src)


def _head_body(r_ref, xg1_ref, xg2_ref, xi1_ref, xi2_ref, xi3_ref,
               wg_ref, bg_ref, wf_ref, bf_ref,
               wg1_ref, bg1_ref, wf1_ref, bf1_ref,
               z_ref, xgall_ref, xg1o_ref, z1_ref):
    hi = lax.Precision.HIGHEST
    xg_all = jnp.maximum(xg1_ref[...], 0.0) + jnp.maximum(xg2_ref[...], 0.0)
    out_g = (
        jnp.dot(xg_all, wg_ref[...], preferred_element_type=jnp.float32, precision=hi)
        + bg_ref[...][None, :]
    )
    xgall_ref[...] = out_g
    z_ref[...] = (
        jnp.dot(out_g, wf_ref[...], preferred_element_type=jnp.float32, precision=hi)
        + bf_ref[...][None, :]
    )
    r3 = r_ref[0]
    r4 = r_ref[1]
    r5 = r_ref[2]
    xin_all = (
        r3 * jnp.maximum(xi1_ref[...], 0.0)
        + r4 * jnp.maximum(xi2_ref[...], 0.0)
        + r5 * jnp.maximum(xi3_ref[...], 0.0)
    )
    out_i = (
        jnp.dot(xin_all, wg1_ref[...], preferred_element_type=jnp.float32, precision=hi)
        + bg1_ref[...][None, :]
    )
    xg1o_ref[...] = out_i
    z1_ref[...] = (
        jnp.dot(out_i, wf1_ref[...], preferred_element_type=jnp.float32, precision=hi)
        + bf1_ref[...][None, :]
    )


def _heads(rvec, xg1, xg2, xi1, xi2, xi3, p):
    d2 = 2 * D
    return pl.pallas_call(
        _head_body,
        in_specs=[pl.BlockSpec(memory_space=pltpu.SMEM)] + [pl.BlockSpec()] * 13,
        out_specs=[pl.BlockSpec()] * 4,
        out_shape=[
            jax.ShapeDtypeStruct((NSEG, 1), jnp.float32),
            jax.ShapeDtypeStruct((NSEG, d2), jnp.float32),
            jax.ShapeDtypeStruct((NSEG, d2), jnp.float32),
            jax.ShapeDtypeStruct((NSEG, 1), jnp.float32),
        ],
    )(rvec, xg1, xg2, xi1, xi2, xi3,
      p['Wg'], p['bg'], p['Wf'], p['bf'],
      p['Wg1'], p['bg1'], p['Wf1'], p['bf1'])


def _aggregate(h, src_pad, dst_pad):
    n = h.shape[0]
    msg = _sc_gather(h, src_pad)
    agg = jnp.zeros((n + 1, D), h.dtype).at[dst_pad].add(msg)
    return agg[:n]


def _readout(x, b):
    m = jax.ops.segment_max(x, b, num_segments=NSEG)
    m = jnp.where(jnp.isfinite(m), m, 0.0)
    s = jax.ops.segment_sum(x, b, num_segments=NSEG)
    cnt = jax.ops.segment_sum(jnp.ones((x.shape[0], 1), x.dtype), b, num_segments=NSEG)
    return jnp.concatenate([m, s / jnp.maximum(cnt, 1.0)], axis=1)


def _branch(h, ei, b, p, ids, nrounds):
    e = ei.shape[1]
    src = jnp.concatenate([ei[0], jnp.zeros((EPAD - e,), jnp.int32)])
    n0 = h.shape[0]
    dst = jnp.concatenate([ei[1], jnp.full((EPAD - e,), n0, jnp.int32)])
    outs = []
    for r in range(nrounds):
        i = ids[r]
        agg = _aggregate(h, src, dst)
        hn = jax.nn.relu((h + agg) @ p['Wc%d' % i] + p['bc%d' % i])
        score = jnp.tanh(hn @ p['wp%d' % i])
        n = hn.shape[0]
        k = int(math.ceil(RATIO * n))
        topv, topi = lax.top_k(score, k)
        h = hn[topi] * topv[:, None]
        b = b[topi]
        pos = jnp.full((n,), -1, dtype=jnp.int32).at[topi].set(
            jnp.arange(k, dtype=jnp.int32))
        e_src, e_dst = src[:e], dst[:e]
        s = pos[e_src]
        d = jnp.where(e_dst == n, -1, pos[jnp.minimum(e_dst, n - 1)])
        valid = (s >= 0) & (d >= 0)
        src = jnp.concatenate([jnp.where(valid, s, 0),
                               jnp.zeros((EPAD - e,), jnp.int32)])
        dst = jnp.concatenate([jnp.where(valid, d, k),
                               jnp.full((EPAD - e,), k, jnp.int32)])
        outs.append(_readout(h, b))
    return outs


def kernel(x, a, edge_attr, edge_index, edge, batch, c, params):
    p = params
    h = x @ p['Wt0'] + p['bt0']
    xg1, xg2 = _branch(h, edge_index, batch, p, [1, 2, 3], nrounds=2)

    h2 = a @ p['Wt1'] + p['bt1']
    xi1, xi2, xi3 = _branch(h2, edge, c, p, [4, 5, 6], nrounds=3)

    rvec = jnp.stack([p['r3'], p['r4'], p['r5']])
    z, xg_all, xg1_out, z1 = _heads(rvec, xg1, xg2, xi1, xi2, xi3, p)
    return (z, xg_all, xg1_out, z1)
